# Initial kernel scaffold; baseline (speedup 1.0000x reference)
import jax
import jax.numpy as jnp
from jax.experimental import pallas as pl


def kernel(x, edge_index, edge_attr, batch, target_esm, node_W, node_b, edge_W, edge_b, tmlp_W1, tmlp_b1, tmlp_W2, tmlp_b2, elin_W, elin_b, nn1_W, nn1_b, nn2_W, nn2_b, gn_w, gn_b, gn_alpha, film_gW, film_gb, film_bW, film_bb, h1_W, h1_b, ln_w, ln_b, hf_gW, hf_gb, hf_bW, hf_bb, h2_W1, h2_b1, h2_W2, h2_b2):
    raise NotImplementedError("write your pallas kernel here")


# speedup vs baseline: 1.7387x; 1.7387x over previous
import functools

import jax
import jax.numpy as jnp
from jax import lax
from jax.experimental import pallas as pl
from jax.experimental.pallas import tpu as pltpu
from jax.experimental.pallas import tpu_sc as plsc

F32 = jnp.float32
G = 64
GP = 72
H = 256
HH = 128
TD = 256
ESM = 640
NP = 10240
EP = 163840
L = 8
NBLK = 512
NSTEPS = NP // NBLK
EBLK = 2048
ESTEPS = EP // EBLK
CH = 128
NTILES = 16
EPW = EP // NTILES
NCH = EPW // CH
STRIPE = NP // NTILES
NRT = NP // NTILES
RCH = 128
NKCH = NRT // RCH
BW = 320
BCH = 64
NBCH = BW // BCH


def _tecnt_body(tesm, w1, b1, w2, b2, pt, te_out, cnt_out):
    t = jax.nn.silu(tesm[...] @ w1[...] + b1[...])
    te_out[...] = t @ w2[...] + b2[...]
    cnt_out[...] = jnp.maximum(jnp.sum(pt[...], axis=1, keepdims=True), 1.0)


def _tecnt(tesm, w1, b1, w2, b2, pt):
    return pl.pallas_call(
        _tecnt_body,
        out_shape=[jax.ShapeDtypeStruct((G, TD), F32),
                   jax.ShapeDtypeStruct((G, 1), F32)],
    )(tesm, w1, b1, w2, b2, pt)


def _layerw_body(fgW, fgb, fbW, fbb, te, gam_out, bet_out):
    t = te[...]
    gam_out[0] = t @ fgW[0] + fgb[0] + 1.0
    bet_out[0] = t @ fbW[0] + fbb[0]


def _layerw(fgW, fgb, fbW, fbb, te):
    full2 = lambda shape: pl.BlockSpec(shape, lambda i: (0, 0))
    per3 = lambda shape: pl.BlockSpec(shape, lambda i: (i, 0, 0))
    return pl.pallas_call(
        _layerw_body,
        grid=(L,),
        in_specs=[per3((1, TD, H)), per3((1, 1, H)),
                  per3((1, TD, H)), per3((1, 1, H)), full2((G, TD))],
        out_specs=[per3((1, G, H)), per3((1, G, H))],
        out_shape=[jax.ShapeDtypeStruct((L, G, H), F32),
                   jax.ShapeDtypeStruct((L, G, H), F32)],
    )(fgW, fgb, fbW, fbb, te)


def _enc_split_body(xb, w, b, out):
    r = xb[...] @ w[...] + b[...]
    out[0] = r[:, :HH]
    out[1] = r[:, HH:]


def _node_enc(x_pad, node_W, node_b):
    return pl.pallas_call(
        _enc_split_body,
        grid=(NSTEPS,),
        in_specs=[pl.BlockSpec((NBLK, H), lambda i: (i, 0)),
                  pl.BlockSpec((H, H), lambda i: (0, 0)),
                  pl.BlockSpec((1, H), lambda i: (0, 0))],
        out_specs=pl.BlockSpec((2, NBLK, HH), lambda i: (0, i, 0)),
        out_shape=jax.ShapeDtypeStruct((2, NP, HH), F32),
    )(x_pad, node_W, node_b)


def _ea_base_body(eb_, w, b, out):
    out[...] = eb_[...] @ w[...] + b[...]


def _ea_base(eap, edge_W, edge_b):
    return pl.pallas_call(
        _ea_base_body,
        grid=(ESTEPS,),
        in_specs=[pl.BlockSpec((EBLK, 16), lambda i: (i, 0)),
                  pl.BlockSpec((16, H), lambda i: (0, 0)),
                  pl.BlockSpec((1, H), lambda i: (0, 0))],
        out_specs=pl.BlockSpec((EBLK, H), lambda i: (i, 0)),
        out_shape=jax.ShapeDtypeStruct((EP, H), F32),
    )(eap, edge_W, edge_b)


def _ea_layer(ea, elinW, elinb):
    return pl.pallas_call(
        _enc_split_body,
        grid=(ESTEPS,),
        in_specs=[pl.BlockSpec((EBLK, H), lambda i: (i, 0)),
                  pl.BlockSpec((H, H), lambda i: (0, 0)),
                  pl.BlockSpec((1, H), lambda i: (0, 0))],
        out_specs=pl.BlockSpec((2, EBLK, HH), lambda i: (0, i, 0)),
        out_shape=jax.ShapeDtypeStruct((2, EP, HH), F32),
    )(ea, elinW, elinb)


def _z_body(hs0, hs1, ag0, ag1, w1, b1, w2, b2, z2_out):
    hcat = jnp.concatenate([hs0[0], hs1[0]], axis=1)
    acat = jnp.concatenate([ag0[0], ag1[0]], axis=1)
    z = hcat + acat
    z = jax.nn.silu(z @ w1[...] + b1[...]) @ w2[...] + b2[...]
    z2_out[0] = z[:, :HH]
    z2_out[1] = z[:, HH:]


def _z_mlp(h_split, aggr_split, w1, b1, w2, b2):
    half = lambda j: pl.BlockSpec((1, NBLK, HH), lambda i, j=j: (j, i, 0))
    return pl.pallas_call(
        _z_body,
        grid=(NSTEPS,),
        in_specs=[half(0), half(1), half(0), half(1),
                  pl.BlockSpec((H, H), lambda i: (0, 0)),
                  pl.BlockSpec((1, H), lambda i: (0, 0)),
                  pl.BlockSpec((H, H), lambda i: (0, 0)),
                  pl.BlockSpec((1, H), lambda i: (0, 0))],
        out_specs=pl.BlockSpec((2, NBLK, HH), lambda i: (0, i, 0)),
        out_shape=jax.ShapeDtypeStruct((2, NP, HH), F32),
    )(h_split, h_split, aggr_split, aggr_split, w1, b1, w2, b2)


def _ad_body(s1, s2, cnt, gam, bet, gnw, gnb, gna, out):
    c = cnt[...]
    s1c = jnp.concatenate([s1[0], s1[1]], axis=1)
    s2c = jnp.concatenate([s2[0], s2[1]], axis=1)
    mean = s1c / c
    var = s2c / c
    sv = jnp.sqrt(var + 1e-5)
    gw = gnw[...]
    gm = gam[...]
    A = gm * gw / sv
    D = gm * (gnb[...] - gw * gna[...] * mean / sv) + bet[...]
    out[:, :H] = A
    out[:, H:] = D


def _ad(s1, s2, cnt, gam, bet, gnw, gnb, gna):
    return pl.pallas_call(
        _ad_body,
        out_shape=jax.ShapeDtypeStruct((GP, 2 * H), F32),
    )(s1, s2, cnt, gam, bet, gnw, gnb, gna)


def _norm_body(z20, z21, adb, hs0, hs1, out):
    ad = adb[...]
    z2 = jnp.concatenate([z20[0], z21[0]], axis=1)
    zc = ad[:, :H] * z2 + ad[:, H:]
    hcat = jnp.concatenate([hs0[0], hs1[0]], axis=1)
    hn = hcat + jax.nn.silu(zc)
    out[0] = hn[:, :HH]
    out[1] = hn[:, HH:]


def _norm(z2_split, adb, h_split):
    half = lambda j: pl.BlockSpec((1, NBLK, HH), lambda i, j=j: (j, i, 0))
    return pl.pallas_call(
        _norm_body,
        grid=(NSTEPS,),
        in_specs=[half(0), half(1),
                  pl.BlockSpec((NBLK, 2 * H), lambda i: (i, 0)),
                  half(0), half(1)],
        out_specs=pl.BlockSpec((2, NBLK, HH), lambda i: (0, i, 0)),
        out_shape=jax.ShapeDtypeStruct((2, NP, HH), F32),
    )(z2_split, z2_split, adb, h_split, h_split)


def _final_body(acc, te, h1W, h1b, lnw, lnb, hfgW, hfgb, hfbW, hfbb,
                h2W1, h2b1, h2W2, h2b2, out):
    a = acc[...]
    ge = jnp.concatenate([a[0:G], a[GP:GP + G]], axis=1)
    y = jax.nn.silu(ge @ h1W[...] + h1b[...])
    mu = jnp.mean(y, axis=-1, keepdims=True)
    v = jnp.mean((y - mu) ** 2, axis=-1, keepdims=True)
    y = (y - mu) / jnp.sqrt(v + 1e-5) * lnw[...] + lnb[...]
    t = te[...]
    gm = t @ hfgW[...] + hfgb[...] + 1.0
    bt = t @ hfbW[...] + hfbb[...]
    y = gm * y + bt
    y = jax.nn.silu(y @ h2W1[...] + h2b1[...]) @ h2W2[...] + h2b2[...]
    out[...] = jnp.maximum(y, 0.0)


def _final(acc, te, h1W, h1b, lnw, lnb, hfgW, hfgb, hfbW, hfbb,
           h2W1, h2b1, h2W2, h2b2):
    return pl.pallas_call(
        _final_body,
        out_shape=jax.ShapeDtypeStruct((G, 1), F32),
    )(acc, te, h1W, h1b, lnw, lnb, hfgW, hfgb, hfbW, hfbb,
      h2W1, h2b1, h2W2, h2b2)


def _sc_edge(h_flat, ea_flat, srcp, dstp, zrows):
    mesh = plsc.VectorSubcoreMesh(core_axis_name="c", subcore_axis_name="s")

    @functools.partial(
        pl.kernel,
        out_type=jax.ShapeDtypeStruct((2 * NP, HH), F32),
        mesh=mesh,
        scratch_types=[
            pltpu.VMEM((CH,), jnp.int32), pltpu.VMEM((CH,), jnp.int32),
            pltpu.VMEM((CH,), jnp.int32),
            pltpu.VMEM((CH, HH), F32), pltpu.VMEM((CH, HH), F32),
            pltpu.VMEM_SHARED((NP, HH), F32),
            pltpu.SemaphoreType.DMA, pltpu.SemaphoreType.DMA,
            pltpu.SemaphoreType.DMA, pltpu.SemaphoreType.DMA,
        ],
    )
    def body(h_hbm, ea_hbm, src_hbm, dst_hbm, z_hbm, out_hbm,
             si0, di0, gi0, ea0, rw0, aggr, ssi, sdi, se, sg):
        c = lax.axis_index("c")
        s = lax.axis_index("s")
        ebase0 = s * EPW
        eaoff = c * EP
        offv = jnp.full((16,), 0, jnp.int32) + c * NP

        pltpu.sync_copy(z_hbm, aggr.at[pl.ds(s * STRIPE, STRIPE)])
        plsc.subcore_barrier()

        def loopbody(k, carry):
            eo = ebase0 + k * CH
            pltpu.async_copy(src_hbm.at[pl.ds(eo, CH)], si0, ssi)
            pltpu.async_copy(dst_hbm.at[pl.ds(eo, CH)], di0, sdi)
            pltpu.async_copy(ea_hbm.at[pl.ds(eaoff + eo, CH)], ea0, se)
            pltpu.make_async_copy(src_hbm.at[pl.ds(eo, CH)], si0, ssi).wait()
            for j in range(CH // 16):
                sl = pl.ds(j * 16, 16)
                gi0[sl] = si0[sl] + offv
            pltpu.async_copy(h_hbm.at[gi0], rw0, sg).wait()
            pltpu.make_async_copy(ea_hbm.at[pl.ds(eaoff + eo, CH)], ea0,
                                  se).wait()

            def rowbody(r, cc):
                for qq in range(HH // 16):
                    sl = pl.ds(qq * 16, 16)
                    rw0[r, sl] = jnp.maximum(rw0[r, sl] + ea0[r, sl], 0.0)
                return cc

            lax.fori_loop(0, CH, rowbody, 0)
            pltpu.make_async_copy(dst_hbm.at[pl.ds(eo, CH)], di0, sdi).wait()
            pltpu.sync_copy(rw0, aggr.at[di0], add=True)
            return carry

        lax.fori_loop(0, NCH, loopbody, 0)

        plsc.subcore_barrier()
        pltpu.sync_copy(aggr.at[pl.ds(s * STRIPE, STRIPE)],
                        out_hbm.at[pl.ds(c * NP + s * STRIPE, STRIPE)])

    return body(h_flat, ea_flat, srcp, dstp, zrows)


def _sc_stats(z2_flat, batchp, cntp, alphav, zgp):
    mesh = plsc.VectorSubcoreMesh(core_axis_name="c", subcore_axis_name="s")

    @functools.partial(
        pl.kernel,
        out_type=[jax.ShapeDtypeStruct((2 * GP, HH), F32),
                  jax.ShapeDtypeStruct((2 * GP, HH), F32)],
        mesh=mesh,
        scratch_types=[
            pltpu.VMEM((RCH, HH), F32), pltpu.VMEM((RCH, HH), F32),
            pltpu.VMEM((RCH,), jnp.int32),
            pltpu.VMEM((RCH + 16,), jnp.int32),
            pltpu.VMEM((GP, HH), F32), pltpu.VMEM((GP + 16,), F32),
            pltpu.VMEM((HH,), F32),
            pltpu.VMEM_SHARED((GP, HH), F32), pltpu.VMEM_SHARED((GP, HH), F32),
            pltpu.SemaphoreType.DMA, pltpu.SemaphoreType.DMA,
            pltpu.SemaphoreType.DMA,
        ],
    )
    def body(z2_hbm, b_hbm, cnt_hbm, al_hbm, zg_hbm, s1_hbm, s2_hbm,
             zbuf, sqbuf, idxb, idxe, meanv, cntv, alv, acc1, acc2,
             sz, si, se2):
        c = lax.axis_index("c")
        s = lax.axis_index("s")
        zoff = c * NP

        @pl.when(s == 0)
        def _():
            pltpu.sync_copy(zg_hbm, acc1)
            pltpu.sync_copy(zg_hbm, acc2)

        plsc.subcore_barrier()

        def chunkA(k, carry):
            ro = s * NRT + k * RCH
            pltpu.async_copy(z2_hbm.at[pl.ds(zoff + ro, RCH)], zbuf, sz)
            pltpu.async_copy(b_hbm.at[pl.ds(ro, RCH)], idxb, si)
            pltpu.make_async_copy(z2_hbm.at[pl.ds(zoff + ro, RCH)], zbuf,
                                  sz).wait()
            pltpu.make_async_copy(b_hbm.at[pl.ds(ro, RCH)], idxb,
                                  si).wait()
            pltpu.sync_copy(zbuf, acc1.at[idxb], add=True)
            return carry

        lax.fori_loop(0, NKCH, chunkA, 0)
        plsc.subcore_barrier()

        pltpu.sync_copy(acc1, meanv)
        pltpu.sync_copy(cnt_hbm, cntv.at[pl.ds(0, GP)])
        pltpu.sync_copy(al_hbm.at[pl.ds(c * HH, HH)], alv)

        def divg(g, carry):
            cv = cntv[pl.ds(g, 16)][0]
            for j in range(HH // 16):
                sl = pl.ds(j * 16, 16)
                meanv[g, sl] = meanv[g, sl] / cv
            return carry

        lax.fori_loop(0, GP, divg, 0)

        def chunkB(k, carry):
            ro = s * NRT + k * RCH
            pltpu.async_copy(z2_hbm.at[pl.ds(zoff + ro, RCH)], zbuf, sz)
            pltpu.async_copy(b_hbm.at[pl.ds(ro, RCH)], idxb, si)
            pltpu.async_copy(b_hbm.at[pl.ds(ro, RCH)],
                             idxe.at[pl.ds(0, RCH)], se2)
            pltpu.make_async_copy(z2_hbm.at[pl.ds(zoff + ro, RCH)], zbuf,
                                  sz).wait()
            pltpu.make_async_copy(b_hbm.at[pl.ds(ro, RCH)], idxb,
                                  si).wait()
            pltpu.make_async_copy(b_hbm.at[pl.ds(ro, RCH)],
                                  idxe.at[pl.ds(0, RCH)], se2).wait()

            def rowb(r, cc):
                g = idxe[pl.ds(r, 16)][0]
                for j in range(HH // 16):
                    sl = pl.ds(j * 16, 16)
                    zc = zbuf[r, sl] - alv[sl] * meanv[g, sl]
                    sqbuf[r, sl] = zc * zc
                return cc

            lax.fori_loop(0, RCH, rowb, 0)
            pltpu.sync_copy(sqbuf, acc2.at[idxb], add=True)
            return carry

        lax.fori_loop(0, NKCH, chunkB, 0)
        plsc.subcore_barrier()

        @pl.when(s == 0)
        def _():
            pltpu.sync_copy(acc1, s1_hbm.at[pl.ds(c * GP, GP)])
            pltpu.sync_copy(acc2, s2_hbm.at[pl.ds(c * GP, GP)])

    return body(z2_flat, batchp, cntp, alphav, zgp)


def _sc_bcast(ad, batchp):
    mesh = plsc.VectorSubcoreMesh(core_axis_name="c", subcore_axis_name="s")

    @functools.partial(
        pl.kernel,
        out_type=jax.ShapeDtypeStruct((NP, 2 * H), F32),
        mesh=mesh,
        scratch_types=[
            pltpu.VMEM((BCH, 2 * H), F32), pltpu.VMEM((BCH,), jnp.int32),
            pltpu.SemaphoreType.DMA, pltpu.SemaphoreType.DMA,
        ],
    )
    def body(ad_hbm, b_hbm, out_hbm, buf, idxb, sg, si):
        c = lax.axis_index("c")
        s = lax.axis_index("s")
        w = s * 2 + c

        def chunk(k, carry):
            ro = w * BW + k * BCH
            pltpu.async_copy(b_hbm.at[pl.ds(ro, BCH)], idxb, si).wait()
            pltpu.async_copy(ad_hbm.at[idxb], buf, sg).wait()
            pltpu.sync_copy(buf, out_hbm.at[pl.ds(ro, BCH)])
            return carry

        lax.fori_loop(0, NBCH, chunk, 0)

    return body(ad, batchp)


def _sc_pool(h_flat, batchp, zgp2):
    mesh = plsc.VectorSubcoreMesh(core_axis_name="c", subcore_axis_name="s")

    @functools.partial(
        pl.kernel,
        out_type=jax.ShapeDtypeStruct((2 * GP, HH), F32),
        mesh=mesh,
        scratch_types=[
            pltpu.VMEM((RCH, HH), F32), pltpu.VMEM((RCH,), jnp.int32),
            pltpu.VMEM((RCH,), jnp.int32),
            pltpu.VMEM_SHARED((2 * GP, HH), F32),
            pltpu.SemaphoreType.DMA, pltpu.SemaphoreType.DMA,
        ],
    )
    def body(h_hbm, b_hbm, zg_hbm, out_hbm, buf, idxb, gidx, acc, sh, si):
        c = lax.axis_index("c")
        s = lax.axis_index("s")

        @pl.when(c == 0)
        def _():
            @pl.when(s == 0)
            def _():
                pltpu.sync_copy(zg_hbm, acc)

            plsc.subcore_barrier()

            for half in range(2):
                offv = jnp.full((16,), 0, jnp.int32) + half * GP

                def chunk(k, carry):
                    ro = s * NRT + k * RCH
                    pltpu.async_copy(h_hbm.at[pl.ds(half * NP + ro, RCH)],
                                     buf, sh)
                    pltpu.async_copy(b_hbm.at[pl.ds(ro, RCH)], idxb, si)
                    pltpu.make_async_copy(h_hbm.at[pl.ds(half * NP + ro, RCH)],
                                          buf, sh).wait()
                    pltpu.make_async_copy(b_hbm.at[pl.ds(ro, RCH)], idxb,
                                          si).wait()
                    for j in range(RCH // 16):
                        sl = pl.ds(j * 16, 16)
                        gidx[sl] = idxb[sl] + offv
                    pltpu.sync_copy(buf, acc.at[gidx], add=True)
                    return carry

                lax.fori_loop(0, NKCH, chunk, 0)

            plsc.subcore_barrier()

            @pl.when(s == 0)
            def _():
                pltpu.sync_copy(acc, out_hbm)

    return body(h_flat, batchp, zgp2)


def kernel(x, edge_index, edge_attr, batch, target_esm, node_W, node_b,
           edge_W, edge_b, tmlp_W1, tmlp_b1, tmlp_W2, tmlp_b2, elin_W,
           elin_b, nn1_W, nn1_b, nn2_W, nn2_b, gn_w, gn_b, gn_alpha,
           film_gW, film_gb, film_bW, film_bb, h1_W, h1_b, ln_w, ln_b,
           hf_gW, hf_gb, hf_bW, hf_bb, h2_W1, h2_b1, h2_W2, h2_b2):
    n = x.shape[0]
    e = edge_index.shape[1]

    x_pad = jnp.pad(x, ((0, NP - n), (0, 0)))
    srcp = jnp.pad(edge_index[0], (0, EP - e))
    dstp = jnp.pad(edge_index[1], (0, EP - e), constant_values=n)
    eap = jnp.pad(edge_attr, ((0, EP - e), (0, 0)))
    batchp = jnp.pad(batch, (0, NP - n), constant_values=G)
    P = (batch[:, None] == jnp.arange(G, dtype=batch.dtype)[None, :]).astype(F32)
    PT = jnp.pad(P, ((0, NP - n), (0, 0))).T
    zrows = jnp.zeros((STRIPE, HH), F32)
    zgp = jnp.zeros((GP, HH), F32)
    zgp2 = jnp.zeros((2 * GP, HH), F32)
    r1 = lambda v: v.reshape(1, -1)
    padg = lambda v: jnp.pad(v, ((0, GP - G), (0, 0)))

    te, cnt = _tecnt(target_esm, tmlp_W1, r1(tmlp_b1), tmlp_W2, r1(tmlp_b2), PT)
    cntp = jnp.pad(cnt, ((0, GP - G), (0, 0)), constant_values=1.0)
    cntp1 = cntp[:, 0]
    gam_all, bet_all = _layerw(
        film_gW, film_gb.reshape(L, 1, H), film_bW, film_bb.reshape(L, 1, H),
        te)

    h_split = _node_enc(x_pad, node_W, r1(node_b))
    ea = _ea_base(eap, edge_W, r1(edge_b))

    for l in range(L):
        ea_split = _ea_layer(ea, elin_W[l], r1(elin_b[l]))
        aggr_flat = _sc_edge(h_split.reshape(2 * NP, HH),
                             ea_split.reshape(2 * EP, HH), srcp, dstp, zrows)
        aggr_split = aggr_flat.reshape(2, NP, HH)
        z2_split = _z_mlp(h_split, aggr_split, nn1_W[l], r1(nn1_b[l]),
                          nn2_W[l], r1(nn2_b[l]))
        s1f, s2f = _sc_stats(z2_split.reshape(2 * NP, HH), batchp, cntp1,
                             gn_alpha[l], zgp)
        adt = _ad(s1f.reshape(2, GP, HH), s2f.reshape(2, GP, HH), cntp,
                  padg(gam_all[l]), padg(bet_all[l]),
                  r1(gn_w[l]), r1(gn_b[l]), r1(gn_alpha[l]))
        adb = _sc_bcast(adt, batchp)
        h_split = _norm(z2_split, adb, h_split)

    acc = _sc_pool(h_split.reshape(2 * NP, HH), batchp, zgp2)
    return _final(acc, te, h1_W, r1(h1_b), r1(ln_w), r1(ln_b),
                  hf_gW, r1(hf_gb), hf_bW, r1(hf_bb),
                  h2_W1, r1(h2_b1), h2_W2, h2_b2.reshape(1, 1))


# speedup vs baseline: 1.7415x; 1.0016x over previous
import functools

import jax
import jax.numpy as jnp
from jax import lax
from jax.experimental import pallas as pl
from jax.experimental.pallas import tpu as pltpu
from jax.experimental.pallas import tpu_sc as plsc

F32 = jnp.float32
G = 64
GP = 72
H = 256
HH = 128
TD = 256
ESM = 640
NP = 10240
EP = 163840
L = 8
NBLK = 512
NSTEPS = NP // NBLK
EBLK = 2048
ESTEPS = EP // EBLK
CH = 128
CH2 = 256
NCH2 = (EP // 16) // CH2
NTILES = 16
EPW = EP // NTILES
NCH = EPW // CH
STRIPE = NP // NTILES
NRT = NP // NTILES
RCH = 128
NKCH = NRT // RCH
BW = 320
BCH = 64
NBCH = BW // BCH


def _tecnt_body(tesm, w1, b1, w2, b2, pt, te_out, cnt_out):
    t = jax.nn.silu(tesm[...] @ w1[...] + b1[...])
    te_out[...] = t @ w2[...] + b2[...]
    cnt_out[...] = jnp.maximum(jnp.sum(pt[...], axis=1, keepdims=True), 1.0)


def _tecnt(tesm, w1, b1, w2, b2, pt):
    return pl.pallas_call(
        _tecnt_body,
        out_shape=[jax.ShapeDtypeStruct((G, TD), F32),
                   jax.ShapeDtypeStruct((G, 1), F32)],
    )(tesm, w1, b1, w2, b2, pt)


def _layerw_body(fgW, fgb, fbW, fbb, te, gam_out, bet_out):
    t = te[...]
    gam_out[0] = t @ fgW[0] + fgb[0] + 1.0
    bet_out[0] = t @ fbW[0] + fbb[0]


def _layerw(fgW, fgb, fbW, fbb, te):
    full2 = lambda shape: pl.BlockSpec(shape, lambda i: (0, 0))
    per3 = lambda shape: pl.BlockSpec(shape, lambda i: (i, 0, 0))
    return pl.pallas_call(
        _layerw_body,
        grid=(L,),
        in_specs=[per3((1, TD, H)), per3((1, 1, H)),
                  per3((1, TD, H)), per3((1, 1, H)), full2((G, TD))],
        out_specs=[per3((1, G, H)), per3((1, G, H))],
        out_shape=[jax.ShapeDtypeStruct((L, G, H), F32),
                   jax.ShapeDtypeStruct((L, G, H), F32)],
    )(fgW, fgb, fbW, fbb, te)


def _enc_split_body(xb, w, b, out):
    r = xb[...] @ w[...] + b[...]
    out[0] = r[:, :HH]
    out[1] = r[:, HH:]


def _node_enc(x_pad, node_W, node_b):
    return pl.pallas_call(
        _enc_split_body,
        grid=(NSTEPS,),
        in_specs=[pl.BlockSpec((NBLK, H), lambda i: (i, 0)),
                  pl.BlockSpec((H, H), lambda i: (0, 0)),
                  pl.BlockSpec((1, H), lambda i: (0, 0))],
        out_specs=pl.BlockSpec((2, NBLK, HH), lambda i: (0, i, 0)),
        out_shape=jax.ShapeDtypeStruct((2, NP, HH), F32),
    )(x_pad, node_W, node_b)


def _ea_base_body(eb_, w, b, out):
    out[...] = eb_[...] @ w[...] + b[...]


def _ea_base(eap, edge_W, edge_b):
    return pl.pallas_call(
        _ea_base_body,
        grid=(ESTEPS,),
        in_specs=[pl.BlockSpec((EBLK, 16), lambda i: (i, 0)),
                  pl.BlockSpec((16, H), lambda i: (0, 0)),
                  pl.BlockSpec((1, H), lambda i: (0, 0))],
        out_specs=pl.BlockSpec((EBLK, H), lambda i: (i, 0)),
        out_shape=jax.ShapeDtypeStruct((EP, H), F32),
    )(eap, edge_W, edge_b)


def _ea_layer(ea, elinW, elinb):
    return pl.pallas_call(
        _enc_split_body,
        grid=(ESTEPS,),
        in_specs=[pl.BlockSpec((EBLK, H), lambda i: (i, 0)),
                  pl.BlockSpec((H, H), lambda i: (0, 0)),
                  pl.BlockSpec((1, H), lambda i: (0, 0))],
        out_specs=pl.BlockSpec((2, EBLK, HH), lambda i: (0, i, 0)),
        out_shape=jax.ShapeDtypeStruct((2, EP, HH), F32),
    )(ea, elinW, elinb)


def _z_body(hs0, hs1, ag0, ag1, w1, b1, w2, b2, z2_out):
    hcat = jnp.concatenate([hs0[0], hs1[0]], axis=1)
    acat = jnp.concatenate([ag0[0], ag1[0]], axis=1)
    z = hcat + acat
    z = jax.nn.silu(z @ w1[...] + b1[...]) @ w2[...] + b2[...]
    z2_out[0] = z[:, :HH]
    z2_out[1] = z[:, HH:]


def _z_mlp(h_split, aggr_split, w1, b1, w2, b2):
    half = lambda j: pl.BlockSpec((1, NBLK, HH), lambda i, j=j: (j, i, 0))
    return pl.pallas_call(
        _z_body,
        grid=(NSTEPS,),
        in_specs=[half(0), half(1), half(0), half(1),
                  pl.BlockSpec((H, H), lambda i: (0, 0)),
                  pl.BlockSpec((1, H), lambda i: (0, 0)),
                  pl.BlockSpec((H, H), lambda i: (0, 0)),
                  pl.BlockSpec((1, H), lambda i: (0, 0))],
        out_specs=pl.BlockSpec((2, NBLK, HH), lambda i: (0, i, 0)),
        out_shape=jax.ShapeDtypeStruct((2, NP, HH), F32),
    )(h_split, h_split, aggr_split, aggr_split, w1, b1, w2, b2)


def _ad_body(s1, s2, cnt, gam, bet, gnw, gnb, gna, out):
    c = cnt[...]
    s1c = jnp.concatenate([s1[0], s1[1]], axis=1)
    s2c = jnp.concatenate([s2[0], s2[1]], axis=1)
    mean = s1c / c
    var = s2c / c
    sv = jnp.sqrt(var + 1e-5)
    gw = gnw[...]
    gm = gam[...]
    A = gm * gw / sv
    D = gm * (gnb[...] - gw * gna[...] * mean / sv) + bet[...]
    out[:, :H] = A
    out[:, H:] = D


def _ad(s1, s2, cnt, gam, bet, gnw, gnb, gna):
    return pl.pallas_call(
        _ad_body,
        out_shape=jax.ShapeDtypeStruct((GP, 2 * H), F32),
    )(s1, s2, cnt, gam, bet, gnw, gnb, gna)


def _norm_body(z20, z21, adb, hs0, hs1, out):
    ad = adb[...]
    z2 = jnp.concatenate([z20[0], z21[0]], axis=1)
    zc = ad[:, :H] * z2 + ad[:, H:]
    hcat = jnp.concatenate([hs0[0], hs1[0]], axis=1)
    hn = hcat + jax.nn.silu(zc)
    out[0] = hn[:, :HH]
    out[1] = hn[:, HH:]


def _norm(z2_split, adb, h_split):
    half = lambda j: pl.BlockSpec((1, NBLK, HH), lambda i, j=j: (j, i, 0))
    return pl.pallas_call(
        _norm_body,
        grid=(NSTEPS,),
        in_specs=[half(0), half(1),
                  pl.BlockSpec((NBLK, 2 * H), lambda i: (i, 0)),
                  half(0), half(1)],
        out_specs=pl.BlockSpec((2, NBLK, HH), lambda i: (0, i, 0)),
        out_shape=jax.ShapeDtypeStruct((2, NP, HH), F32),
    )(z2_split, z2_split, adb, h_split, h_split)


def _final_body(acc, te, h1W, h1b, lnw, lnb, hfgW, hfgb, hfbW, hfbb,
                h2W1, h2b1, h2W2, h2b2, out):
    a = acc[...]
    ge = jnp.concatenate([a[0:G], a[GP:GP + G]], axis=1)
    y = jax.nn.silu(ge @ h1W[...] + h1b[...])
    mu = jnp.mean(y, axis=-1, keepdims=True)
    v = jnp.mean((y - mu) ** 2, axis=-1, keepdims=True)
    y = (y - mu) / jnp.sqrt(v + 1e-5) * lnw[...] + lnb[...]
    t = te[...]
    gm = t @ hfgW[...] + hfgb[...] + 1.0
    bt = t @ hfbW[...] + hfbb[...]
    y = gm * y + bt
    y = jax.nn.silu(y @ h2W1[...] + h2b1[...]) @ h2W2[...] + h2b2[...]
    out[...] = jnp.maximum(y, 0.0)


def _final(acc, te, h1W, h1b, lnw, lnb, hfgW, hfgb, hfbW, hfbb,
           h2W1, h2b1, h2W2, h2b2):
    return pl.pallas_call(
        _final_body,
        out_shape=jax.ShapeDtypeStruct((G, 1), F32),
    )(acc, te, h1W, h1b, lnw, lnb, hfgW, hfgb, hfbW, hfbb,
      h2W1, h2b1, h2W2, h2b2)


def _sc_edge(h_flat, ea_flat, srcp, dstp, zrows):
    mesh = plsc.VectorSubcoreMesh(core_axis_name="c", subcore_axis_name="s")

    @functools.partial(
        pl.kernel,
        out_type=jax.ShapeDtypeStruct((2 * NP, HH), F32),
        mesh=mesh,
        scratch_types=[
            pltpu.VMEM((CH,), jnp.int32), pltpu.VMEM((CH,), jnp.int32),
            pltpu.VMEM((CH,), jnp.int32),
            pltpu.VMEM((CH, HH), F32), pltpu.VMEM((CH, HH), F32),
            pltpu.VMEM_SHARED((NP, HH), F32),
            pltpu.SemaphoreType.DMA, pltpu.SemaphoreType.DMA,
            pltpu.SemaphoreType.DMA, pltpu.SemaphoreType.DMA,
        ],
    )
    def body(h_hbm, ea_hbm, src_hbm, dst_hbm, z_hbm, out_hbm,
             si0, di0, gi0, ea0, rw0, aggr, ssi, sdi, se, sg):
        c = lax.axis_index("c")
        s = lax.axis_index("s")
        ebase0 = s * EPW
        eaoff = c * EP
        offv = jnp.full((16,), 0, jnp.int32) + c * NP

        pltpu.sync_copy(z_hbm, aggr.at[pl.ds(s * STRIPE, STRIPE)])
        plsc.subcore_barrier()

        def loopbody(k, carry):
            eo = ebase0 + k * CH
            pltpu.async_copy(src_hbm.at[pl.ds(eo, CH)], si0, ssi)
            pltpu.async_copy(dst_hbm.at[pl.ds(eo, CH)], di0, sdi)
            pltpu.async_copy(ea_hbm.at[pl.ds(eaoff + eo, CH)], ea0, se)
            pltpu.make_async_copy(src_hbm.at[pl.ds(eo, CH)], si0, ssi).wait()
            for j in range(CH // 16):
                sl = pl.ds(j * 16, 16)
                gi0[sl] = si0[sl] + offv
            pltpu.async_copy(h_hbm.at[gi0], rw0, sg).wait()
            pltpu.make_async_copy(ea_hbm.at[pl.ds(eaoff + eo, CH)], ea0,
                                  se).wait()

            def rowbody(r, cc):
                for qq in range(HH // 16):
                    sl = pl.ds(qq * 16, 16)
                    rw0[r, sl] = jnp.maximum(rw0[r, sl] + ea0[r, sl], 0.0)
                return cc

            lax.fori_loop(0, CH, rowbody, 0)
            pltpu.make_async_copy(dst_hbm.at[pl.ds(eo, CH)], di0, sdi).wait()
            pltpu.sync_copy(rw0, aggr.at[di0], add=True)
            return carry

        lax.fori_loop(0, NCH, loopbody, 0)

        plsc.subcore_barrier()
        pltpu.sync_copy(aggr.at[pl.ds(s * STRIPE, STRIPE)],
                        out_hbm.at[pl.ds(c * NP + s * STRIPE, STRIPE)])

    return body(h_flat, ea_flat, srcp, dstp, zrows)


def _sc_stats(z2_flat, batchp, cntp, alphav, zgp):
    mesh = plsc.VectorSubcoreMesh(core_axis_name="c", subcore_axis_name="s")

    @functools.partial(
        pl.kernel,
        out_type=[jax.ShapeDtypeStruct((2 * GP, HH), F32),
                  jax.ShapeDtypeStruct((2 * GP, HH), F32)],
        mesh=mesh,
        scratch_types=[
            pltpu.VMEM((RCH, HH), F32), pltpu.VMEM((RCH, HH), F32),
            pltpu.VMEM((RCH,), jnp.int32),
            pltpu.VMEM((RCH + 16,), jnp.int32),
            pltpu.VMEM((GP, HH), F32), pltpu.VMEM((GP + 16,), F32),
            pltpu.VMEM((HH,), F32),
            pltpu.VMEM_SHARED((GP, HH), F32), pltpu.VMEM_SHARED((GP, HH), F32),
            pltpu.SemaphoreType.DMA, pltpu.SemaphoreType.DMA,
            pltpu.SemaphoreType.DMA,
        ],
    )
    def body(z2_hbm, b_hbm, cnt_hbm, al_hbm, zg_hbm, s1_hbm, s2_hbm,
             zbuf, sqbuf, idxb, idxe, meanv, cntv, alv, acc1, acc2,
             sz, si, se2):
        c = lax.axis_index("c")
        s = lax.axis_index("s")
        zoff = c * NP

        @pl.when(s == 0)
        def _():
            pltpu.sync_copy(zg_hbm, acc1)
            pltpu.sync_copy(zg_hbm, acc2)

        plsc.subcore_barrier()

        def chunkA(k, carry):
            ro = s * NRT + k * RCH
            pltpu.async_copy(z2_hbm.at[pl.ds(zoff + ro, RCH)], zbuf, sz)
            pltpu.async_copy(b_hbm.at[pl.ds(ro, RCH)], idxb, si)
            pltpu.make_async_copy(z2_hbm.at[pl.ds(zoff + ro, RCH)], zbuf,
                                  sz).wait()
            pltpu.make_async_copy(b_hbm.at[pl.ds(ro, RCH)], idxb,
                                  si).wait()
            pltpu.sync_copy(zbuf, acc1.at[idxb], add=True)
            return carry

        lax.fori_loop(0, NKCH, chunkA, 0)
        plsc.subcore_barrier()

        pltpu.sync_copy(acc1, meanv)
        pltpu.sync_copy(cnt_hbm, cntv.at[pl.ds(0, GP)])
        pltpu.sync_copy(al_hbm.at[pl.ds(c * HH, HH)], alv)

        def divg(g, carry):
            cv = cntv[pl.ds(g, 16)][0]
            for j in range(HH // 16):
                sl = pl.ds(j * 16, 16)
                meanv[g, sl] = meanv[g, sl] / cv
            return carry

        lax.fori_loop(0, GP, divg, 0)

        def chunkB(k, carry):
            ro = s * NRT + k * RCH
            pltpu.async_copy(z2_hbm.at[pl.ds(zoff + ro, RCH)], zbuf, sz)
            pltpu.async_copy(b_hbm.at[pl.ds(ro, RCH)], idxb, si)
            pltpu.async_copy(b_hbm.at[pl.ds(ro, RCH)],
                             idxe.at[pl.ds(0, RCH)], se2)
            pltpu.make_async_copy(z2_hbm.at[pl.ds(zoff + ro, RCH)], zbuf,
                                  sz).wait()
            pltpu.make_async_copy(b_hbm.at[pl.ds(ro, RCH)], idxb,
                                  si).wait()
            pltpu.make_async_copy(b_hbm.at[pl.ds(ro, RCH)],
                                  idxe.at[pl.ds(0, RCH)], se2).wait()

            def rowb(r, cc):
                g = idxe[pl.ds(r, 16)][0]
                for j in range(HH // 16):
                    sl = pl.ds(j * 16, 16)
                    zc = zbuf[r, sl] - alv[sl] * meanv[g, sl]
                    sqbuf[r, sl] = zc * zc
                return cc

            lax.fori_loop(0, RCH, rowb, 0)
            pltpu.sync_copy(sqbuf, acc2.at[idxb], add=True)
            return carry

        lax.fori_loop(0, NKCH, chunkB, 0)
        plsc.subcore_barrier()

        @pl.when(s == 0)
        def _():
            pltpu.sync_copy(acc1, s1_hbm.at[pl.ds(c * GP, GP)])
            pltpu.sync_copy(acc2, s2_hbm.at[pl.ds(c * GP, GP)])

    return body(z2_flat, batchp, cntp, alphav, zgp)


def _sc_bcast(ad, batchp):
    mesh = plsc.VectorSubcoreMesh(core_axis_name="c", subcore_axis_name="s")

    @functools.partial(
        pl.kernel,
        out_type=jax.ShapeDtypeStruct((NP, 2 * H), F32),
        mesh=mesh,
        scratch_types=[
            pltpu.VMEM((BCH, 2 * H), F32), pltpu.VMEM((BCH,), jnp.int32),
            pltpu.SemaphoreType.DMA, pltpu.SemaphoreType.DMA,
        ],
    )
    def body(ad_hbm, b_hbm, out_hbm, buf, idxb, sg, si):
        c = lax.axis_index("c")
        s = lax.axis_index("s")
        w = s * 2 + c

        def chunk(k, carry):
            ro = w * BW + k * BCH
            pltpu.async_copy(b_hbm.at[pl.ds(ro, BCH)], idxb, si).wait()
            pltpu.async_copy(ad_hbm.at[idxb], buf, sg).wait()
            pltpu.sync_copy(buf, out_hbm.at[pl.ds(ro, BCH)])
            return carry

        lax.fori_loop(0, NBCH, chunk, 0)

    return body(ad, batchp)


def _sc_pool(h_flat, batchp, zgp2):
    mesh = plsc.VectorSubcoreMesh(core_axis_name="c", subcore_axis_name="s")

    @functools.partial(
        pl.kernel,
        out_type=jax.ShapeDtypeStruct((2 * GP, HH), F32),
        mesh=mesh,
        scratch_types=[
            pltpu.VMEM((RCH, HH), F32), pltpu.VMEM((RCH,), jnp.int32),
            pltpu.VMEM((RCH,), jnp.int32),
            pltpu.VMEM_SHARED((2 * GP, HH), F32),
            pltpu.SemaphoreType.DMA, pltpu.SemaphoreType.DMA,
        ],
    )
    def body(h_hbm, b_hbm, zg_hbm, out_hbm, buf, idxb, gidx, acc, sh, si):
        c = lax.axis_index("c")
        s = lax.axis_index("s")

        @pl.when(c == 0)
        def _():
            @pl.when(s == 0)
            def _():
                pltpu.sync_copy(zg_hbm, acc)

            plsc.subcore_barrier()

            for half in range(2):
                offv = jnp.full((16,), 0, jnp.int32) + half * GP

                def chunk(k, carry):
                    ro = s * NRT + k * RCH
                    pltpu.async_copy(h_hbm.at[pl.ds(half * NP + ro, RCH)],
                                     buf, sh)
                    pltpu.async_copy(b_hbm.at[pl.ds(ro, RCH)], idxb, si)
                    pltpu.make_async_copy(h_hbm.at[pl.ds(half * NP + ro, RCH)],
                                          buf, sh).wait()
                    pltpu.make_async_copy(b_hbm.at[pl.ds(ro, RCH)], idxb,
                                          si).wait()
                    for j in range(RCH // 16):
                        sl = pl.ds(j * 16, 16)
                        gidx[sl] = idxb[sl] + offv
                    pltpu.sync_copy(buf, acc.at[gidx], add=True)
                    return carry

                lax.fori_loop(0, NKCH, chunk, 0)

            plsc.subcore_barrier()

            @pl.when(s == 0)
            def _():
                pltpu.sync_copy(acc, out_hbm)

    return body(h_flat, batchp, zgp2)


def kernel(x, edge_index, edge_attr, batch, target_esm, node_W, node_b,
           edge_W, edge_b, tmlp_W1, tmlp_b1, tmlp_W2, tmlp_b2, elin_W,
           elin_b, nn1_W, nn1_b, nn2_W, nn2_b, gn_w, gn_b, gn_alpha,
           film_gW, film_gb, film_bW, film_bb, h1_W, h1_b, ln_w, ln_b,
           hf_gW, hf_gb, hf_bW, hf_bb, h2_W1, h2_b1, h2_W2, h2_b2):
    n = x.shape[0]
    e = edge_index.shape[1]

    x_pad = jnp.pad(x, ((0, NP - n), (0, 0)))
    srcp = jnp.pad(edge_index[0], (0, EP - e))
    dstp = jnp.pad(edge_index[1], (0, EP - e), constant_values=n)
    eap = jnp.pad(edge_attr, ((0, EP - e), (0, 0)))
    batchp = jnp.pad(batch, (0, NP - n), constant_values=G)
    P = (batch[:, None] == jnp.arange(G, dtype=batch.dtype)[None, :]).astype(F32)
    PT = jnp.pad(P, ((0, NP - n), (0, 0))).T
    zrows = jnp.zeros((STRIPE, HH), F32)
    zgp = jnp.zeros((GP, HH), F32)
    zgp2 = jnp.zeros((2 * GP, HH), F32)
    r1 = lambda v: v.reshape(1, -1)
    padg = lambda v: jnp.pad(v, ((0, GP - G), (0, 0)))

    te, cnt = _tecnt(target_esm, tmlp_W1, r1(tmlp_b1), tmlp_W2, r1(tmlp_b2), PT)
    cntp = jnp.pad(cnt, ((0, GP - G), (0, 0)), constant_values=1.0)
    cntp1 = cntp[:, 0]
    gam_all, bet_all = _layerw(
        film_gW, film_gb.reshape(L, 1, H), film_bW, film_bb.reshape(L, 1, H),
        te)

    h_split = _node_enc(x_pad, node_W, r1(node_b))
    ea = _ea_base(eap, edge_W, r1(edge_b))

    for l in range(L):
        ea_split = _ea_layer(ea, elin_W[l], r1(elin_b[l]))
        aggr_flat = _sc_edge(h_split.reshape(2 * NP, HH),
                             ea_split.reshape(2 * EP, HH), srcp, dstp, zrows)
        aggr_split = aggr_flat.reshape(2, NP, HH)
        z2_split = _z_mlp(h_split, aggr_split, nn1_W[l], r1(nn1_b[l]),
                          nn2_W[l], r1(nn2_b[l]))
        s1f, s2f = _sc_stats(z2_split.reshape(2 * NP, HH), batchp, cntp1,
                             gn_alpha[l], zgp)
        adt = _ad(s1f.reshape(2, GP, HH), s2f.reshape(2, GP, HH), cntp,
                  padg(gam_all[l]), padg(bet_all[l]),
                  r1(gn_w[l]), r1(gn_b[l]), r1(gn_alpha[l]))
        adb = _sc_bcast(adt, batchp)
        h_split = _norm(z2_split, adb, h_split)

    acc = _sc_pool(h_split.reshape(2 * NP, HH), batchp, zgp2)
    return _final(acc, te, h1_W, r1(h1_b), r1(ln_w), r1(ln_b),
                  hf_gW, r1(hf_gb), hf_bW, r1(hf_bb),
                  h2_W1, r1(h2_b1), h2_W2, h2_b2.reshape(1, 1))
